# Initial kernel scaffold; baseline (speedup 1.0000x reference)
import jax
import jax.numpy as jnp
from jax.experimental import pallas as pl


def kernel(input_ids, embedding_table):
    raise NotImplementedError("write your pallas kernel here")


# speedup vs baseline: 1.5609x; 1.5609x over previous
import jax
import jax.numpy as jnp
from jax import lax
from jax.experimental import pallas as pl
from jax.experimental.pallas import tpu as pltpu
from jax.experimental.pallas import tpu_sc as plsc
import functools

VOCAB = 100000
D = 1024
B = 4 * 4096
NC, NS = 2, 16
NW = NC * NS
B_PER_W = B // NW
C = 32
NCHUNK = B_PER_W // C


@functools.partial(
    pl.kernel,
    out_type=jax.ShapeDtypeStruct((B, D), jnp.float32),
    mesh=plsc.VectorSubcoreMesh(
        core_axis_name="c", subcore_axis_name="s", num_cores=NC, num_subcores=NS
    ),
    scratch_types=[
        pltpu.VMEM((NCHUNK, C), jnp.int32),
        pltpu.VMEM((C, D), jnp.float32),
        pltpu.VMEM((C, D), jnp.float32),
        pltpu.SemaphoreType.DMA,
        pltpu.SemaphoreType.DMA,
        pltpu.SemaphoreType.DMA,
        pltpu.SemaphoreType.DMA,
    ],
)
def _embed_sc(idx_hbm, table_hbm, out_hbm, idx_v, buf0, buf1, g0, g1, o0, o1):
    wid = lax.axis_index("s") * NC + lax.axis_index("c")
    base = wid * B_PER_W
    pltpu.sync_copy(idx_hbm.at[wid], idx_v)

    bufs = (buf0, buf1)
    gsem = (g0, g1)
    osem = (o0, o1)
    gather = [None, None]
    outcp = [None, None]

    gather[0] = pltpu.async_copy(table_hbm.at[idx_v.at[0]], bufs[0], gsem[0])
    for c in range(NCHUNK):
        b = c % 2
        gather[b].wait()
        if c + 1 < NCHUNK:
            nb = (c + 1) % 2
            if outcp[nb] is not None:
                outcp[nb].wait()
            gather[nb] = pltpu.async_copy(
                table_hbm.at[idx_v.at[c + 1]], bufs[nb], gsem[nb]
            )
        outcp[b] = pltpu.async_copy(
            bufs[b], out_hbm.at[pl.ds(base + c * C, C)], osem[b]
        )
    outcp[0].wait()
    outcp[1].wait()


def kernel(input_ids, embedding_table):
    idx = input_ids.reshape(NW, NCHUNK, C).astype(jnp.int32)
    out = _embed_sc(idx, embedding_table)
    return out.reshape(input_ids.shape + (D,))
